# Initial kernel scaffold; baseline (speedup 1.0000x reference)
import jax
import jax.numpy as jnp
from jax.experimental import pallas as pl


def kernel(x, edge_index, W1, b1, W2, b2):
    raise NotImplementedError("write your pallas kernel here")


# speedup vs baseline: 21.5122x; 21.5122x over previous
import functools

import jax
import jax.numpy as jnp
from jax import lax
from jax.experimental import pallas as pl
from jax.experimental.pallas import tpu as pltpu
from jax.experimental.pallas import tpu_sc as plsc

NC = 2
NS = 16
NW = NC * NS
CB = 128


def _make_deg(np_rows, k_chunks):
    rpt = np_rows // NS
    mesh = plsc.VectorSubcoreMesh(core_axis_name="c", subcore_axis_name="s")

    @functools.partial(
        pl.kernel,
        out_type=jax.ShapeDtypeStruct((NC, np_rows, 16), jnp.float32),
        mesh=mesh,
        compiler_params=pltpu.CompilerParams(use_tc_tiling_on_sc=False),
        scratch_types=[
            pltpu.VMEM((k_chunks, CB), jnp.int32),
            pltpu.VMEM((CB, 16), jnp.float32),
            pltpu.VMEM_SHARED((np_rows, 16), jnp.float32),
        ],
    )
    def deg_kernel(dst_hbm, zeros_hbm, ones_hbm, out_hbm, dst_v, ones_v, acc):
        c = lax.axis_index("c")
        s = lax.axis_index("s")
        wid = c * NS + s
        pltpu.sync_copy(zeros_hbm.at[pl.ds(s * rpt, rpt)],
                        acc.at[pl.ds(s * rpt, rpt)])
        pltpu.sync_copy(dst_hbm.at[wid], dst_v)
        pltpu.sync_copy(ones_hbm, ones_v)
        plsc.subcore_barrier()

        def body(j, carry):
            pltpu.sync_copy(ones_v, acc.at[dst_v.at[j]], add=True)
            return carry

        lax.fori_loop(0, k_chunks, body, 0)
        plsc.subcore_barrier()
        pltpu.sync_copy(acc.at[pl.ds(s * rpt, rpt)],
                        out_hbm.at[c, pl.ds(s * rpt, rpt)])

    return deg_kernel


def _make_prop(np_rows, d, k_chunks):
    rpt = np_rows // NS
    mesh = plsc.VectorSubcoreMesh(core_axis_name="c", subcore_axis_name="s")

    @functools.partial(
        pl.kernel,
        out_type=jax.ShapeDtypeStruct((NC, np_rows, d), jnp.float32),
        mesh=mesh,
        compiler_params=pltpu.CompilerParams(use_tc_tiling_on_sc=False),
        scratch_types=[
            pltpu.VMEM((k_chunks, CB), jnp.int32),
            pltpu.VMEM((k_chunks, CB), jnp.int32),
            pltpu.VMEM((CB, d), jnp.float32),
            pltpu.VMEM_SHARED((np_rows, d), jnp.float32),
            pltpu.SemaphoreType.DMA,
        ],
    )
    def prop_kernel(src_hbm, dst_hbm, g_hbm, zeros_hbm, out_hbm,
                    src_v, dst_v, rows_v, acc, sem):
        c = lax.axis_index("c")
        s = lax.axis_index("s")
        wid = c * NS + s
        pltpu.sync_copy(zeros_hbm.at[pl.ds(s * rpt, rpt)],
                        acc.at[pl.ds(s * rpt, rpt)])
        pltpu.sync_copy(src_hbm.at[wid], src_v)
        pltpu.sync_copy(dst_hbm.at[wid], dst_v)
        plsc.subcore_barrier()

        def body(j, carry):
            pltpu.async_copy(g_hbm.at[src_v.at[j]], rows_v, sem).wait()
            pltpu.sync_copy(rows_v, acc.at[dst_v.at[j]], add=True)
            return carry

        lax.fori_loop(0, k_chunks, body, 0)
        plsc.subcore_barrier()
        pltpu.sync_copy(acc.at[pl.ds(s * rpt, rpt)],
                        out_hbm.at[c, pl.ds(s * rpt, rpt)])

    return prop_kernel


def _prescale_body(degp_ref, x_ref, g1_ref):
    deg = degp_ref[0, :, 0:1] + degp_ref[1, :, 0:1]
    g1_ref[...] = x_ref[...] * lax.rsqrt(deg)


def _mlp_body(degp_ref, s1p_ref, w1_ref, b1_ref, w2_ref, g2_ref):
    dinv = lax.rsqrt(degp_ref[0, :, 0:1] + degp_ref[1, :, 0:1])
    a1 = (s1p_ref[0] + s1p_ref[1]) * dinv
    h = jnp.dot(a1, w1_ref[...], preferred_element_type=jnp.float32)
    h = jnp.maximum(h + b1_ref[...], 0.0)
    t = jnp.dot(h, w2_ref[...], preferred_element_type=jnp.float32)
    g2_ref[...] = t * dinv


def _final_body(degp_ref, s2p_ref, b2_ref, out_ref):
    dinv = lax.rsqrt(degp_ref[0, :, 0:1] + degp_ref[1, :, 0:1])
    out_ref[...] = (s2p_ref[0] + s2p_ref[1]) * dinv + b2_ref[...]


def kernel(x, edge_index, W1, b1, W2, b2):
    n, d = x.shape
    h_dim = W1.shape[1]
    c_dim = W2.shape[1]
    e = edge_index.shape[1]

    loops = jnp.arange(n, dtype=jnp.int32)
    e_all = e + n
    k_chunks = -(-e_all // (NW * CB))
    pad = NW * k_chunks * CB - e_all
    src = jnp.concatenate([edge_index[0], loops,
                           jnp.zeros((pad,), jnp.int32)]).reshape(NW, k_chunks, CB)
    dst = jnp.concatenate([edge_index[1], loops,
                           jnp.full((pad,), n, jnp.int32)]).reshape(NW, k_chunks, CB)

    np_rows = -(-(n + 1) // (NS * 8)) * NS * 8

    z16 = jnp.zeros((np_rows, 16), jnp.float32)
    zd = jnp.zeros((np_rows, d), jnp.float32)
    zc = jnp.zeros((np_rows, c_dim), jnp.float32)
    ones = jnp.ones((CB, 16), jnp.float32)

    degp = _make_deg(np_rows, k_chunks)(dst, z16, ones)[:, :n]

    bn = 1000
    grid = (n // bn,)
    deg_spec = pl.BlockSpec((2, bn, 16), lambda i: (0, i, 0))

    g1 = pl.pallas_call(
        _prescale_body,
        grid=grid,
        in_specs=[deg_spec, pl.BlockSpec((bn, d), lambda i: (i, 0))],
        out_specs=pl.BlockSpec((bn, d), lambda i: (i, 0)),
        out_shape=jax.ShapeDtypeStruct((n, d), jnp.float32),
    )(degp, x)

    s1p = _make_prop(np_rows, d, k_chunks)(src, dst, g1, zd)[:, :n]

    g2 = pl.pallas_call(
        _mlp_body,
        grid=grid,
        in_specs=[
            deg_spec,
            pl.BlockSpec((2, bn, d), lambda i: (0, i, 0)),
            pl.BlockSpec((d, h_dim), lambda i: (0, 0)),
            pl.BlockSpec((1, h_dim), lambda i: (0, 0)),
            pl.BlockSpec((h_dim, c_dim), lambda i: (0, 0)),
        ],
        out_specs=pl.BlockSpec((bn, c_dim), lambda i: (i, 0)),
        out_shape=jax.ShapeDtypeStruct((n, c_dim), jnp.float32),
    )(degp, s1p, W1, b1.reshape(1, h_dim), W2)

    s2p = _make_prop(np_rows, c_dim, k_chunks)(src, dst, g2, zc)[:, :n]

    out = pl.pallas_call(
        _final_body,
        grid=grid,
        in_specs=[
            deg_spec,
            pl.BlockSpec((2, bn, c_dim), lambda i: (0, i, 0)),
            pl.BlockSpec((1, c_dim), lambda i: (0, 0)),
        ],
        out_specs=pl.BlockSpec((bn, c_dim), lambda i: (i, 0)),
        out_shape=jax.ShapeDtypeStruct((n, c_dim), jnp.float32),
    )(degp, s2p, b2.reshape(1, c_dim))

    return out


# speedup vs baseline: 28.7719x; 1.3375x over previous
import functools

import jax
import jax.numpy as jnp
from jax import lax
from jax.experimental import pallas as pl
from jax.experimental.pallas import tpu as pltpu
from jax.experimental.pallas import tpu_sc as plsc

NC = 2
NS = 16
NW = NC * NS
CB = 128
NB = 4


def _make_deg(np_rows, k_chunks):
    rpt = np_rows // NS
    mesh = plsc.VectorSubcoreMesh(core_axis_name="c", subcore_axis_name="s")

    @functools.partial(
        pl.kernel,
        out_type=jax.ShapeDtypeStruct((NC, np_rows, 16), jnp.float32),
        mesh=mesh,
        compiler_params=pltpu.CompilerParams(use_tc_tiling_on_sc=False),
        scratch_types=[
            pltpu.VMEM((k_chunks, CB), jnp.int32),
            pltpu.VMEM((CB, 16), jnp.float32),
            pltpu.VMEM_SHARED((np_rows, 16), jnp.float32),
        ],
    )
    def deg_kernel(dst_hbm, zeros_hbm, ones_hbm, out_hbm, dst_v, ones_v, acc):
        c = lax.axis_index("c")
        s = lax.axis_index("s")
        wid = c * NS + s
        pltpu.sync_copy(zeros_hbm.at[pl.ds(s * rpt, rpt)],
                        acc.at[pl.ds(s * rpt, rpt)])
        pltpu.sync_copy(dst_hbm.at[wid], dst_v)
        pltpu.sync_copy(ones_hbm, ones_v)
        plsc.subcore_barrier()

        def body(j, carry):
            pltpu.sync_copy(ones_v, acc.at[dst_v.at[j]], add=True)
            return carry

        lax.fori_loop(0, k_chunks, body, 0)
        plsc.subcore_barrier()
        pltpu.sync_copy(acc.at[pl.ds(s * rpt, rpt)],
                        out_hbm.at[c, pl.ds(s * rpt, rpt)])

    return deg_kernel


def _make_prop(np_rows, dh, k_chunks):
    rpt = np_rows // NS
    mesh = plsc.VectorSubcoreMesh(core_axis_name="c", subcore_axis_name="s")

    @functools.partial(
        pl.kernel,
        out_type=jax.ShapeDtypeStruct((NC, np_rows, dh), jnp.float32),
        mesh=mesh,
        compiler_params=pltpu.CompilerParams(use_tc_tiling_on_sc=False),
        scratch_types=[
            pltpu.VMEM((k_chunks, CB), jnp.int32),
            pltpu.VMEM((k_chunks, CB), jnp.int32),
            pltpu.VMEM((NB, CB, dh), jnp.float32),
            pltpu.VMEM_SHARED((np_rows, dh), jnp.float32),
            pltpu.SemaphoreType.DMA,
            pltpu.SemaphoreType.DMA,
        ],
    )
    def prop_kernel(src_hbm, dst_hbm, g_hbm, zeros_hbm, out_hbm,
                    src_v, dst_v, rows_v, acc, gsem, ssem):
        c = lax.axis_index("c")
        s = lax.axis_index("s")
        gh = g_hbm.at[c]
        pltpu.sync_copy(zeros_hbm.at[pl.ds(s * rpt, rpt)],
                        acc.at[pl.ds(s * rpt, rpt)])
        pltpu.sync_copy(src_hbm.at[s], src_v)
        pltpu.sync_copy(dst_hbm.at[s], dst_v)
        plsc.subcore_barrier()

        for b in range(NB):
            pltpu.async_copy(gh.at[src_v.at[b]], rows_v.at[b], gsem)

        def body(j, carry):
            bj = lax.rem(j, NB)
            pltpu.make_async_copy(gh.at[src_v.at[bj]],
                                  rows_v.at[bj], gsem).wait()
            pltpu.async_copy(rows_v.at[bj], acc.at[dst_v.at[j]], ssem,
                             add=True)
            nxt = j + (NB - 1)

            @pl.when((j >= 1) & (nxt < k_chunks))
            def _():
                bp = lax.rem(nxt, NB)
                pltpu.make_async_copy(rows_v.at[bp],
                                      acc.at[dst_v.at[0]], ssem).wait()
                pltpu.async_copy(gh.at[src_v.at[nxt]], rows_v.at[bp], gsem)

            return carry

        lax.fori_loop(0, k_chunks, body, 0)
        for _ in range(NB):
            pltpu.make_async_copy(rows_v.at[0], acc.at[dst_v.at[0]],
                                  ssem).wait()
        plsc.subcore_barrier()
        pltpu.sync_copy(acc.at[pl.ds(s * rpt, rpt)],
                        out_hbm.at[c, pl.ds(s * rpt, rpt)])

    return prop_kernel


def _prescale_body(degp_ref, x_ref, g1_ref):
    deg = degp_ref[0, :, 0:1] + degp_ref[1, :, 0:1]
    dinv = lax.rsqrt(deg)
    d = x_ref.shape[1]
    g1_ref[0] = x_ref[:, : d // 2] * dinv
    g1_ref[1] = x_ref[:, d // 2:] * dinv


def _mlp_body(degp_ref, s1_ref, w1_ref, b1_ref, w2_ref, g2_ref):
    dinv = lax.rsqrt(degp_ref[0, :, 0:1] + degp_ref[1, :, 0:1])
    a1 = jnp.concatenate([s1_ref[0], s1_ref[1]], axis=1) * dinv
    h = jnp.dot(a1, w1_ref[...], preferred_element_type=jnp.float32)
    h = jnp.maximum(h + b1_ref[...], 0.0)
    t = jnp.dot(h, w2_ref[...], preferred_element_type=jnp.float32)
    g2 = t * dinv
    ch = t.shape[1] // 2
    g2_ref[0] = g2[:, :ch]
    g2_ref[1] = g2[:, ch:]


def _final_body(degp_ref, s2p_ref, b2_ref, out_ref):
    dinv = lax.rsqrt(degp_ref[0, :, 0:1] + degp_ref[1, :, 0:1])
    s2 = jnp.concatenate([s2p_ref[0], s2p_ref[1]], axis=1)
    out_ref[...] = s2 * dinv + b2_ref[...]


def kernel(x, edge_index, W1, b1, W2, b2):
    n, d = x.shape
    h_dim = W1.shape[1]
    c_dim = W2.shape[1]
    e = edge_index.shape[1]

    loops = jnp.arange(n, dtype=jnp.int32)
    e_all = e + n
    k_prop = -(-e_all // (NS * CB))
    e_pad = NS * k_prop * CB
    pad = e_pad - e_all
    src = jnp.concatenate([edge_index[0], loops, jnp.zeros((pad,), jnp.int32)])
    dst = jnp.concatenate([edge_index[1], loops, jnp.full((pad,), n, jnp.int32)])
    k_deg = e_pad // (NW * CB)
    src_p = src.reshape(NS, k_prop, CB)
    dst_p = dst.reshape(NS, k_prop, CB)
    dst_d = dst.reshape(NW, k_deg, CB)

    np_rows = -(-(n + 1) // (NS * 8)) * NS * 8

    z16 = jnp.zeros((np_rows, 16), jnp.float32)
    zd = jnp.zeros((np_rows, d // 2), jnp.float32)
    zc = jnp.zeros((np_rows, c_dim // 2), jnp.float32)
    ones = jnp.ones((CB, 16), jnp.float32)

    degp = _make_deg(np_rows, k_deg)(dst_d, z16, ones)[:, :n]

    bn = 1000
    grid = (n // bn,)
    deg_spec = pl.BlockSpec((2, bn, 16), lambda i: (0, i, 0))

    g1 = pl.pallas_call(
        _prescale_body,
        grid=grid,
        in_specs=[deg_spec, pl.BlockSpec((bn, d), lambda i: (i, 0))],
        out_specs=pl.BlockSpec((2, bn, d // 2), lambda i: (0, i, 0)),
        out_shape=jax.ShapeDtypeStruct((2, n, d // 2), jnp.float32),
    )(degp, x)

    s1 = _make_prop(np_rows, d // 2, k_prop)(src_p, dst_p, g1, zd)[:, :n]

    g2 = pl.pallas_call(
        _mlp_body,
        grid=grid,
        in_specs=[
            deg_spec,
            pl.BlockSpec((2, bn, d // 2), lambda i: (0, i, 0)),
            pl.BlockSpec((d, h_dim), lambda i: (0, 0)),
            pl.BlockSpec((1, h_dim), lambda i: (0, 0)),
            pl.BlockSpec((h_dim, c_dim), lambda i: (0, 0)),
        ],
        out_specs=pl.BlockSpec((2, bn, c_dim // 2), lambda i: (0, i, 0)),
        out_shape=jax.ShapeDtypeStruct((2, n, c_dim // 2), jnp.float32),
    )(degp, s1, W1, b1.reshape(1, h_dim), W2)

    s2 = _make_prop(np_rows, c_dim // 2, k_prop)(src_p, dst_p, g2, zc)[:, :n]

    out = pl.pallas_call(
        _final_body,
        grid=grid,
        in_specs=[
            deg_spec,
            pl.BlockSpec((2, bn, c_dim // 2), lambda i: (0, i, 0)),
            pl.BlockSpec((1, c_dim), lambda i: (0, 0)),
        ],
        out_specs=pl.BlockSpec((bn, c_dim), lambda i: (i, 0)),
        out_shape=jax.ShapeDtypeStruct((n, c_dim), jnp.float32),
    )(degp, s2, b2.reshape(1, c_dim))

    return out


# speedup vs baseline: 29.2084x; 1.0152x over previous
import functools

import jax
import jax.numpy as jnp
from jax import lax
from jax.experimental import pallas as pl
from jax.experimental.pallas import tpu as pltpu
from jax.experimental.pallas import tpu_sc as plsc

NC = 2
NS = 16
NW = NC * NS
CB = 128
NB = 4


def _make_deg(np_rows, k_chunks):
    rpt = np_rows // NS
    mesh = plsc.VectorSubcoreMesh(core_axis_name="c", subcore_axis_name="s")

    @functools.partial(
        pl.kernel,
        out_type=jax.ShapeDtypeStruct((NC, np_rows, 16), jnp.float32),
        mesh=mesh,
        compiler_params=pltpu.CompilerParams(use_tc_tiling_on_sc=False),
        scratch_types=[
            pltpu.VMEM((k_chunks, CB), jnp.int32),
            pltpu.VMEM((CB, 16), jnp.float32),
            pltpu.VMEM_SHARED((np_rows, 16), jnp.float32),
        ],
    )
    def deg_kernel(dst_hbm, zeros_hbm, ones_hbm, out_hbm, dst_v, ones_v, acc):
        c = lax.axis_index("c")
        s = lax.axis_index("s")
        wid = c * NS + s
        pltpu.sync_copy(zeros_hbm.at[pl.ds(s * rpt, rpt)],
                        acc.at[pl.ds(s * rpt, rpt)])
        pltpu.sync_copy(dst_hbm.at[wid], dst_v)
        pltpu.sync_copy(ones_hbm, ones_v)
        plsc.subcore_barrier()

        def body(j, carry):
            pltpu.sync_copy(ones_v, acc.at[dst_v.at[j]], add=True)
            return carry

        lax.fori_loop(0, k_chunks, body, 0)
        plsc.subcore_barrier()
        pltpu.sync_copy(acc.at[pl.ds(s * rpt, rpt)],
                        out_hbm.at[c, pl.ds(s * rpt, rpt)])

    return deg_kernel


def _make_prop(np_rows, dh, k_chunks):
    rpt = np_rows // NS
    mesh = plsc.VectorSubcoreMesh(core_axis_name="c", subcore_axis_name="s")

    @functools.partial(
        pl.kernel,
        out_type=jax.ShapeDtypeStruct((NC, np_rows, dh), jnp.float32),
        mesh=mesh,
        compiler_params=pltpu.CompilerParams(use_tc_tiling_on_sc=False),
        scratch_types=[
            pltpu.VMEM((k_chunks, CB), jnp.int32),
            pltpu.VMEM((k_chunks, CB), jnp.int32),
            pltpu.VMEM((NB, CB, dh), jnp.float32),
            pltpu.VMEM_SHARED((np_rows, dh), jnp.float32),
            pltpu.SemaphoreType.DMA,
            pltpu.SemaphoreType.DMA,
        ],
    )
    def prop_kernel(src_hbm, dst_hbm, g_hbm, zeros_hbm, out_hbm,
                    src_v, dst_v, rows_v, acc, gsem, ssem):
        c = lax.axis_index("c")
        s = lax.axis_index("s")
        gh = g_hbm.at[c]
        pltpu.sync_copy(zeros_hbm.at[pl.ds(s * rpt, rpt)],
                        acc.at[pl.ds(s * rpt, rpt)])
        pltpu.sync_copy(src_hbm.at[s], src_v)
        pltpu.sync_copy(dst_hbm.at[s], dst_v)
        plsc.subcore_barrier()

        for b in range(NB):
            pltpu.async_copy(gh.at[src_v.at[b]], rows_v.at[b], gsem)

        def body(j, carry):
            bj = lax.rem(j, NB)
            pltpu.make_async_copy(gh.at[src_v.at[bj]],
                                  rows_v.at[bj], gsem).wait()
            pltpu.async_copy(rows_v.at[bj], acc.at[dst_v.at[j]], ssem,
                             add=True)
            nxt = j + (NB - 1)

            @pl.when((j >= 1) & (nxt < k_chunks))
            def _():
                bp = lax.rem(nxt, NB)
                pltpu.make_async_copy(rows_v.at[bp],
                                      acc.at[dst_v.at[0]], ssem).wait()
                pltpu.async_copy(gh.at[src_v.at[nxt]], rows_v.at[bp], gsem)

            return carry

        lax.fori_loop(0, k_chunks, body, 0)
        for _ in range(NB):
            pltpu.make_async_copy(rows_v.at[0], acc.at[dst_v.at[0]],
                                  ssem).wait()
        plsc.subcore_barrier()
        pltpu.sync_copy(acc.at[pl.ds(s * rpt, rpt)],
                        out_hbm.at[c, pl.ds(s * rpt, rpt)])

    return prop_kernel


def _make_prop_final(np_rows, dh, k_chunks):
    rpt = np_rows // NS
    mesh = plsc.VectorSubcoreMesh(core_axis_name="c", subcore_axis_name="s")

    @functools.partial(
        pl.kernel,
        out_type=jax.ShapeDtypeStruct((NC, np_rows, dh), jnp.float32),
        mesh=mesh,
        compiler_params=pltpu.CompilerParams(use_tc_tiling_on_sc=False),
        scratch_types=[
            pltpu.VMEM((k_chunks, CB), jnp.int32),
            pltpu.VMEM((k_chunks, CB), jnp.int32),
            pltpu.VMEM((NB, CB, dh), jnp.float32),
            pltpu.VMEM((rpt, dh), jnp.float32),
            pltpu.VMEM((rpt, 16), jnp.float32),
            pltpu.VMEM((dh,), jnp.float32),
            pltpu.VMEM_SHARED((np_rows, dh), jnp.float32),
            pltpu.SemaphoreType.DMA,
            pltpu.SemaphoreType.DMA,
        ],
    )
    def prop_kernel(src_hbm, dst_hbm, g_hbm, zeros_hbm, dinv_hbm, bias_hbm,
                    out_hbm, src_v, dst_v, rows_v, res_v, dinv_v, bias_v,
                    acc, gsem, ssem):
        c = lax.axis_index("c")
        s = lax.axis_index("s")
        gh = g_hbm.at[c]
        pltpu.sync_copy(zeros_hbm.at[pl.ds(s * rpt, rpt)],
                        acc.at[pl.ds(s * rpt, rpt)])
        pltpu.sync_copy(src_hbm.at[s], src_v)
        pltpu.sync_copy(dst_hbm.at[s], dst_v)
        pltpu.sync_copy(dinv_hbm.at[pl.ds(s * rpt, rpt)], dinv_v)
        pltpu.sync_copy(bias_hbm.at[c], bias_v)
        plsc.subcore_barrier()

        for b in range(NB):
            pltpu.async_copy(gh.at[src_v.at[b]], rows_v.at[b], gsem)

        def body(j, carry):
            bj = lax.rem(j, NB)
            pltpu.make_async_copy(gh.at[src_v.at[bj]],
                                  rows_v.at[bj], gsem).wait()
            pltpu.async_copy(rows_v.at[bj], acc.at[dst_v.at[j]], ssem,
                             add=True)
            nxt = j + (NB - 1)

            @pl.when((j >= 1) & (nxt < k_chunks))
            def _():
                bp = lax.rem(nxt, NB)
                pltpu.make_async_copy(rows_v.at[bp],
                                      acc.at[dst_v.at[0]], ssem).wait()
                pltpu.async_copy(gh.at[src_v.at[nxt]], rows_v.at[bp], gsem)

            return carry

        lax.fori_loop(0, k_chunks, body, 0)
        for _ in range(NB):
            pltpu.make_async_copy(rows_v.at[0], acc.at[dst_v.at[0]],
                                  ssem).wait()
        plsc.subcore_barrier()
        pltpu.sync_copy(acc.at[pl.ds(s * rpt, rpt)], res_v)
        bias = bias_v[:]

        def fin(i, carry):
            res_v[i, :] = res_v[i, :] * dinv_v[i, :] + bias
            return carry

        lax.fori_loop(0, rpt, fin, 0)
        pltpu.sync_copy(res_v, out_hbm.at[c, pl.ds(s * rpt, rpt)])

    return prop_kernel


def _prescale_body(degp_ref, x_ref, g1_ref, dinv_ref):
    deg = degp_ref[0, :, 0:1] + degp_ref[1, :, 0:1]
    dinv = lax.rsqrt(deg)
    d = x_ref.shape[1]
    g1_ref[0] = x_ref[:, : d // 2] * dinv
    g1_ref[1] = x_ref[:, d // 2:] * dinv
    dinv_ref[...] = jnp.broadcast_to(dinv, dinv_ref.shape)


def _mlp_body(dinv16_ref, s1_ref, w1_ref, b1_ref, w2_ref, g2_ref):
    dinv = dinv16_ref[:, 0:1]
    a1 = jnp.concatenate([s1_ref[0], s1_ref[1]], axis=1) * dinv
    h = jnp.dot(a1, w1_ref[...], preferred_element_type=jnp.float32)
    h = jnp.maximum(h + b1_ref[...], 0.0)
    t = jnp.dot(h, w2_ref[...], preferred_element_type=jnp.float32)
    g2 = t * dinv
    ch = t.shape[1] // 2
    g2_ref[0] = g2[:, :ch]
    g2_ref[1] = g2[:, ch:]


def kernel(x, edge_index, W1, b1, W2, b2):
    n, d = x.shape
    h_dim = W1.shape[1]
    c_dim = W2.shape[1]
    e = edge_index.shape[1]

    loops = jnp.arange(n, dtype=jnp.int32)
    e_all = e + n
    k_prop = -(-e_all // (NS * CB))
    e_pad = NS * k_prop * CB
    pad = e_pad - e_all
    src = jnp.concatenate([edge_index[0], loops, jnp.zeros((pad,), jnp.int32)])
    dst = jnp.concatenate([edge_index[1], loops, jnp.full((pad,), n, jnp.int32)])
    k_deg = e_pad // (NW * CB)
    src_p = src.reshape(NS, k_prop, CB)
    dst_p = dst.reshape(NS, k_prop, CB)
    dst_d = dst.reshape(NW, k_deg, CB)

    np_rows = -(-(n + 1) // (NS * 8)) * NS * 8

    z16 = jnp.zeros((np_rows, 16), jnp.float32)
    zd = jnp.zeros((np_rows, d // 2), jnp.float32)
    zc = jnp.zeros((np_rows, c_dim // 2), jnp.float32)
    ones = jnp.ones((CB, 16), jnp.float32)

    degp = _make_deg(np_rows, k_deg)(dst_d, z16, ones)[:, :n]

    bn = 1000
    grid = (n // bn,)
    deg_spec = pl.BlockSpec((2, bn, 16), lambda i: (0, i, 0))

    g1, dinv16 = pl.pallas_call(
        _prescale_body,
        grid=grid,
        in_specs=[deg_spec, pl.BlockSpec((bn, d), lambda i: (i, 0))],
        out_specs=[
            pl.BlockSpec((2, bn, d // 2), lambda i: (0, i, 0)),
            pl.BlockSpec((bn, 16), lambda i: (i, 0)),
        ],
        out_shape=[
            jax.ShapeDtypeStruct((2, n, d // 2), jnp.float32),
            jax.ShapeDtypeStruct((n, 16), jnp.float32),
        ],
    )(degp, x)

    s1 = _make_prop(np_rows, d // 2, k_prop)(src_p, dst_p, g1, zd)[:, :n]

    g2 = pl.pallas_call(
        _mlp_body,
        grid=grid,
        in_specs=[
            pl.BlockSpec((bn, 16), lambda i: (i, 0)),
            pl.BlockSpec((2, bn, d // 2), lambda i: (0, i, 0)),
            pl.BlockSpec((d, h_dim), lambda i: (0, 0)),
            pl.BlockSpec((1, h_dim), lambda i: (0, 0)),
            pl.BlockSpec((h_dim, c_dim), lambda i: (0, 0)),
        ],
        out_specs=pl.BlockSpec((2, bn, c_dim // 2), lambda i: (0, i, 0)),
        out_shape=jax.ShapeDtypeStruct((2, n, c_dim // 2), jnp.float32),
    )(dinv16, s1, W1, b1.reshape(1, h_dim), W2)

    dinv16p = jnp.pad(dinv16, ((0, np_rows - n), (0, 0)))
    b2h = b2.reshape(NC, c_dim // 2)
    outh = _make_prop_final(np_rows, c_dim // 2, k_prop)(
        src_p, dst_p, g2, zc, dinv16p, b2h)
    return jnp.concatenate([outh[0, :n], outh[1, :n]], axis=1)
